# baseline (speedup 1.0000x reference)
import functools

import jax
import jax.numpy as jnp
from jax import lax
from jax.experimental import pallas as pl
from jax.experimental.pallas import tpu as pltpu
from jax.experimental.pallas import tpu_sc as plsc

_CK = 8


def _sc_body(t_hbm, i_hbm, a_hbm, e_hbm, out_hbm, ids_hbm,
             ib0, ib1, ob0, ob1, bias_v, ids_v,
             isem0, isem1, osem0, osem1,
             *, B, H, seg_lens, tot, nw, nc):
    cid = lax.axis_index("c")
    sid = lax.axis_index("s")
    wid = sid * nc + cid

    ibufs, obufs = (ib0, ib1), (ob0, ob1)
    isems, osems = (isem0, isem1), (osem0, osem1)

    hbms = (t_hbm, i_hbm, a_hbm)
    off = 0
    for m, lm in enumerate(seg_lens):
        x_hbm = hbms[m]
        rm = B * lm // nw
        in_base = wid * rm
        b = in_base // lm
        l0 = in_base - b * lm
        out_base = b * tot + off + l0
        nck = rm // _CK

        pltpu.sync_copy(e_hbm.at[m, :], bias_v)

        for i in range(rm // 16):
            ids_v[pl.ds(i * 16, 16)] = jnp.full((16,), m, jnp.int32)
        pltpu.sync_copy(ids_v.at[pl.ds(0, rm)], ids_hbm.at[pl.ds(out_base, rm)])

        def in_cp(kk, s, in_base=in_base):
            return pltpu.make_async_copy(
                x_hbm.at[pl.ds(in_base + kk * _CK, _CK), :], ibufs[s], isems[s])

        def out_cp(kk, s, out_base=out_base):
            return pltpu.make_async_copy(
                obufs[s], out_hbm.at[pl.ds(out_base + kk * _CK, _CK), :], osems[s])

        for s in (0, 1):
            in_cp(s, s).start()

        def group(g, _):
            for s in (0, 1):
                kk = 2 * g + s
                in_cp(kk, s).wait()

                @pl.when(g > 0)
                def _():
                    out_cp(kk - 2, s).wait()

                def row(r, _, s=s):
                    for c in range(H // 16):
                        sl = pl.ds(c * 16, 16)
                        obufs[s][r, sl] = ibufs[s][r, sl] + bias_v[sl]
                    return 0

                lax.fori_loop(0, _CK, row, 0)
                out_cp(kk, s).start()

                @pl.when(kk + 2 < nck)
                def _():
                    in_cp(kk + 2, s).start()
            return 0

        lax.fori_loop(0, nck // 2, group, 0)
        for s in (0, 1):
            out_cp(nck - 2 + s, s).wait()
        off += lm


def kernel(text, image, audio, modality_embed):
    B, l_t, H = text.shape
    l_i = image.shape[1]
    l_a = audio.shape[1]
    tot = l_t + l_i + l_a

    info = plsc.get_sparse_core_info()
    nc, ns = info.num_cores, info.num_subcores
    nw = nc * ns
    mesh = plsc.VectorSubcoreMesh(core_axis_name="c", subcore_axis_name="s")

    body = functools.partial(_sc_body, B=B, H=H, seg_lens=(l_t, l_i, l_a),
                             tot=tot, nw=nw, nc=nc)

    sck = pl.kernel(
        body,
        mesh=mesh,
        out_type=[
            jax.ShapeDtypeStruct((B * tot, H), jnp.float32),
            jax.ShapeDtypeStruct((B * tot,), jnp.int32),
        ],
        scratch_types=[
            pltpu.VMEM((_CK, H), jnp.float32),
            pltpu.VMEM((_CK, H), jnp.float32),
            pltpu.VMEM((_CK, H), jnp.float32),
            pltpu.VMEM((_CK, H), jnp.float32),
            pltpu.VMEM((H,), jnp.float32),
            pltpu.VMEM((B * l_t // nw,), jnp.int32),
            pltpu.SemaphoreType.DMA,
            pltpu.SemaphoreType.DMA,
            pltpu.SemaphoreType.DMA,
            pltpu.SemaphoreType.DMA,
        ],
    )
    out2, ids1 = sck(
        text.reshape(B * l_t, H),
        image.reshape(B * l_i, H),
        audio.reshape(B * l_a, H),
        modality_embed,
    )
    return out2.reshape(B, tot, H), ids1.reshape(B, tot)


# speedup vs baseline: 2.7384x; 2.7384x over previous
import functools

import jax
import jax.numpy as jnp
from jax import lax
from jax.experimental import pallas as pl
from jax.experimental.pallas import tpu as pltpu
from jax.experimental.pallas import tpu_sc as plsc

_CK = 8


def _sc_body(t_hbm, i_hbm, a_hbm, e_hbm, out_hbm, ids_hbm,
             ib0, ib1, ob0, ob1, bias_v, ids_v,
             isem0, isem1, osem0, osem1,
             *, B, H, seg_lens, tot, nw, nc):
    cid = lax.axis_index("c")
    sid = lax.axis_index("s")
    wid = sid * nc + cid

    ibufs, obufs = (ib0, ib1), (ob0, ob1)
    isems, osems = (isem0, isem1), (osem0, osem1)

    hbms = (t_hbm, i_hbm, a_hbm)
    off = 0
    for m, lm in enumerate(seg_lens):
        x_hbm = hbms[m]
        rm = B * lm // nw
        in_base = wid * rm
        b = in_base // lm
        l0 = in_base - b * lm
        out_base = b * tot + off + l0
        nck = rm // _CK

        pltpu.sync_copy(e_hbm.at[m, :], bias_v)

        for i in range(rm // 16):
            ids_v[pl.ds(i * 16, 16)] = jnp.full((16,), m, jnp.int32)
        pltpu.sync_copy(ids_v.at[pl.ds(0, rm)], ids_hbm.at[pl.ds(out_base, rm)])

        def in_cp(kk, s, in_base=in_base):
            return pltpu.make_async_copy(
                x_hbm.at[pl.ds(in_base + kk * _CK, _CK), :], ibufs[s], isems[s])

        def out_cp(kk, s, out_base=out_base):
            return pltpu.make_async_copy(
                obufs[s], out_hbm.at[pl.ds(out_base + kk * _CK, _CK), :], osems[s])

        for s in (0, 1):
            in_cp(s, s).start()

        def group(g, _):
            for s in (0, 1):
                kk = 2 * g + s
                in_cp(kk, s).wait()

                @pl.when(g > 0)
                def _():
                    out_cp(kk - 2, s).wait()

                @plsc.parallel_loop(0, H // 16, unroll=4)
                def _(c, s=s):
                    sl = pl.ds(c * 16, 16)
                    bv = bias_v[sl]
                    for r in range(_CK):
                        obufs[s][r, sl] = ibufs[s][r, sl] + bv
                out_cp(kk, s).start()

                @pl.when(kk + 2 < nck)
                def _():
                    in_cp(kk + 2, s).start()
            return 0

        lax.fori_loop(0, nck // 2, group, 0)
        for s in (0, 1):
            out_cp(nck - 2 + s, s).wait()
        off += lm


def kernel(text, image, audio, modality_embed):
    B, l_t, H = text.shape
    l_i = image.shape[1]
    l_a = audio.shape[1]
    tot = l_t + l_i + l_a

    info = plsc.get_sparse_core_info()
    nc, ns = info.num_cores, info.num_subcores
    nw = nc * ns
    mesh = plsc.VectorSubcoreMesh(core_axis_name="c", subcore_axis_name="s")

    body = functools.partial(_sc_body, B=B, H=H, seg_lens=(l_t, l_i, l_a),
                             tot=tot, nw=nw, nc=nc)

    sck = pl.kernel(
        body,
        mesh=mesh,
        out_type=[
            jax.ShapeDtypeStruct((B * tot, H), jnp.float32),
            jax.ShapeDtypeStruct((B * tot,), jnp.int32),
        ],
        scratch_types=[
            pltpu.VMEM((_CK, H), jnp.float32),
            pltpu.VMEM((_CK, H), jnp.float32),
            pltpu.VMEM((_CK, H), jnp.float32),
            pltpu.VMEM((_CK, H), jnp.float32),
            pltpu.VMEM((H,), jnp.float32),
            pltpu.VMEM((B * l_t // nw,), jnp.int32),
            pltpu.SemaphoreType.DMA,
            pltpu.SemaphoreType.DMA,
            pltpu.SemaphoreType.DMA,
            pltpu.SemaphoreType.DMA,
        ],
    )
    out2, ids1 = sck(
        text.reshape(B * l_t, H),
        image.reshape(B * l_i, H),
        audio.reshape(B * l_a, H),
        modality_embed,
    )
    return out2.reshape(B, tot, H), ids1.reshape(B, tot)


# speedup vs baseline: 2.8177x; 1.0289x over previous
import functools

import jax
import jax.numpy as jnp
from jax import lax
from jax.experimental import pallas as pl
from jax.experimental.pallas import tpu as pltpu
from jax.experimental.pallas import tpu_sc as plsc

_CK = 8


def _sc_body(t_hbm, i_hbm, a_hbm, e_hbm, out_hbm, ids_hbm,
             ib0, ib1, ob0, ob1, bias_v, ids_v,
             isem0, isem1, osem0, osem1, idsem,
             *, B, H, seg_lens, tot, nw, nc):
    cid = lax.axis_index("c")
    sid = lax.axis_index("s")
    wid = sid * nc + cid

    ibufs, obufs = (ib0, ib1), (ob0, ob1)
    isems, osems = (isem0, isem1), (osem0, osem1)

    pltpu.sync_copy(e_hbm.at[pl.ds(0, len(seg_lens)), :], bias_v)

    hbms = (t_hbm, i_hbm, a_hbm)
    ids_cps = []
    off = 0
    for m, lm in enumerate(seg_lens):
        x_hbm = hbms[m]
        rm = B * lm // nw
        in_base = wid * rm
        b = in_base // lm
        l0 = in_base - b * lm
        out_base = b * tot + off + l0
        nck = rm // _CK

        ids_off = sum(B * l // nw for l in seg_lens[:m])
        for i in range(rm // 16):
            ids_v[pl.ds(ids_off + i * 16, 16)] = jnp.full((16,), m, jnp.int32)
        cp = pltpu.make_async_copy(ids_v.at[pl.ds(ids_off, rm)],
                                   ids_hbm.at[pl.ds(out_base, rm)], idsem)
        cp.start()
        ids_cps.append(cp)

        def in_cp(kk, s, in_base=in_base):
            return pltpu.make_async_copy(
                x_hbm.at[pl.ds(in_base + kk * _CK, _CK), :], ibufs[s], isems[s])

        def out_cp(kk, s, out_base=out_base):
            return pltpu.make_async_copy(
                obufs[s], out_hbm.at[pl.ds(out_base + kk * _CK, _CK), :], osems[s])

        for s in (0, 1):
            in_cp(s, s).start()

        def group(g, _):
            for s in (0, 1):
                kk = 2 * g + s
                in_cp(kk, s).wait()

                @pl.when(g > 0)
                def _():
                    out_cp(kk - 2, s).wait()

                @plsc.parallel_loop(0, H // 16, unroll=8)
                def _(c, s=s, m=m):
                    sl = pl.ds(c * 16, 16)
                    bv = bias_v[m, sl]
                    for r in range(_CK):
                        obufs[s][r, sl] = ibufs[s][r, sl] + bv
                out_cp(kk, s).start()

                @pl.when(kk + 2 < nck)
                def _():
                    in_cp(kk + 2, s).start()
            return 0

        lax.fori_loop(0, nck // 2, group, 0)
        for s in (0, 1):
            out_cp(nck - 2 + s, s).wait()
        off += lm
    for cp in ids_cps:
        cp.wait()


def kernel(text, image, audio, modality_embed):
    B, l_t, H = text.shape
    l_i = image.shape[1]
    l_a = audio.shape[1]
    tot = l_t + l_i + l_a

    info = plsc.get_sparse_core_info()
    nc, ns = info.num_cores, info.num_subcores
    nw = nc * ns
    mesh = plsc.VectorSubcoreMesh(core_axis_name="c", subcore_axis_name="s")

    body = functools.partial(_sc_body, B=B, H=H, seg_lens=(l_t, l_i, l_a),
                             tot=tot, nw=nw, nc=nc)

    sck = pl.kernel(
        body,
        mesh=mesh,
        out_type=[
            jax.ShapeDtypeStruct((B * tot, H), jnp.float32),
            jax.ShapeDtypeStruct((B * tot,), jnp.int32),
        ],
        scratch_types=[
            pltpu.VMEM((_CK, H), jnp.float32),
            pltpu.VMEM((_CK, H), jnp.float32),
            pltpu.VMEM((_CK, H), jnp.float32),
            pltpu.VMEM((_CK, H), jnp.float32),
            pltpu.VMEM((3, H), jnp.float32),
            pltpu.VMEM((B * tot // nw,), jnp.int32),
            pltpu.SemaphoreType.DMA,
            pltpu.SemaphoreType.DMA,
            pltpu.SemaphoreType.DMA,
            pltpu.SemaphoreType.DMA,
            pltpu.SemaphoreType.DMA,
        ],
    )
    out2, ids1 = sck(
        text.reshape(B * l_t, H),
        image.reshape(B * l_i, H),
        audio.reshape(B * l_a, H),
        modality_embed,
    )
    return out2.reshape(B, tot, H), ids1.reshape(B, tot)


# speedup vs baseline: 3.3687x; 1.1956x over previous
import functools

import jax
import jax.numpy as jnp
from jax import lax
from jax.experimental import pallas as pl
from jax.experimental.pallas import tpu as pltpu
from jax.experimental.pallas import tpu_sc as plsc

_C = 512


def _tc_body(t_ref, i_ref, a_ref, emb_ref, out_ref, *, n_t, n_i):
    j = pl.program_id(1)

    @pl.when(j < n_t)
    def _():
        out_ref[...] = t_ref[...] + emb_ref[0, :][None, None, :]

    @pl.when((j >= n_t) & (j < n_t + n_i))
    def _():
        out_ref[...] = i_ref[...] + emb_ref[1, :][None, None, :]

    @pl.when(j >= n_t + n_i)
    def _():
        out_ref[...] = a_ref[...] + emb_ref[2, :][None, None, :]


def _sc_ids_body(ids_hbm, ids_v, *, B, seg_lens, tot, nw, nc):
    cid = lax.axis_index("c")
    sid = lax.axis_index("s")
    wid = sid * nc + cid

    off = 0
    for m, lm in enumerate(seg_lens):
        rm = B * lm // nw
        base = wid * rm
        b = base // lm
        col0 = off + (base - b * lm)
        ids_off = sum(B * l // nw for l in seg_lens[:m])
        for i in range(rm // 16):
            ids_v[pl.ds(ids_off + i * 16, 16)] = jnp.full((16,), m, jnp.int32)
        pltpu.sync_copy(ids_v.at[pl.ds(ids_off, rm)],
                        ids_hbm.at[b, pl.ds(col0, rm)])
        off += lm


def kernel(text, image, audio, modality_embed):
    B, l_t, H = text.shape
    l_i = image.shape[1]
    l_a = audio.shape[1]
    tot = l_t + l_i + l_a
    n_t, n_i, n_a = l_t // _C, l_i // _C, l_a // _C

    info = plsc.get_sparse_core_info()
    nc, ns = info.num_cores, info.num_subcores
    nw = nc * ns
    mesh = plsc.VectorSubcoreMesh(core_axis_name="c", subcore_axis_name="s")

    ids = pl.kernel(
        functools.partial(_sc_ids_body, B=B, seg_lens=(l_t, l_i, l_a),
                          tot=tot, nw=nw, nc=nc),
        mesh=mesh,
        out_type=[jax.ShapeDtypeStruct((B, tot), jnp.int32)],
        scratch_types=[pltpu.VMEM((B * tot // nw,), jnp.int32)],
    )()[0]

    out = pl.pallas_call(
        functools.partial(_tc_body, n_t=n_t, n_i=n_i),
        grid=(B, n_t + n_i + n_a),
        in_specs=[
            pl.BlockSpec((1, _C, H), lambda b, j: (b, jnp.minimum(j, n_t - 1), 0)),
            pl.BlockSpec((1, _C, H), lambda b, j: (b, jnp.clip(j - n_t, 0, n_i - 1), 0)),
            pl.BlockSpec((1, _C, H), lambda b, j: (b, jnp.clip(j - n_t - n_i, 0, n_a - 1), 0)),
            pl.BlockSpec(modality_embed.shape, lambda b, j: (0, 0)),
        ],
        out_specs=pl.BlockSpec((1, _C, H), lambda b, j: (b, j, 0)),
        out_shape=jax.ShapeDtypeStruct((B, tot, H), jnp.float32),
    )(text, image, audio, modality_embed)

    return out, ids
